# baseline (speedup 1.0000x reference)
import functools

import jax
import jax.numpy as jnp
from jax import lax
from jax.experimental import pallas as pl
from jax.experimental.pallas import tpu as pltpu
from jax.experimental.pallas import tpu_sc as plsc

NC = 2
NS = 16
NW = NC * NS
LANES = 16
NBUF = 8
K = 4


def _tc_prep(prompt_no, indices, lp, vocab):
    n, l, d = prompt_no.shape
    batch = indices.shape[0]

    def body(p_ref, i_ref, pn_ref, ip_ref):
        pn = jnp.mean(p_ref[...], axis=0)
        pn_ref[...] = jnp.concatenate(
            [pn, jnp.zeros((lp - l, d), jnp.float32)], axis=0)
        pad = (lax.broadcasted_iota(jnp.int32, (batch, lp - l), 0)
               * (lp - l)
               + lax.broadcasted_iota(jnp.int32, (batch, lp - l), 1))
        ip_ref[...] = jnp.concatenate(
            [i_ref[...], (pad * 127) % vocab], axis=1)

    return pl.pallas_call(
        body,
        out_shape=(
            jax.ShapeDtypeStruct((lp, d), jnp.float32),
            jax.ShapeDtypeStruct((batch, lp), jnp.int32),
        ),
    )(prompt_no, indices)


def _tc_cast(rows, batch, l, lp, d):
    bb = 16

    def body(r_ref, o_ref):
        o_ref[...] = r_ref[...][:, :l, :]

    return pl.pallas_call(
        body,
        grid=(batch // bb,),
        in_specs=[pl.BlockSpec((bb, lp, d), lambda i: (i, 0, 0))],
        out_specs=pl.BlockSpec((bb, l, d), lambda i: (i, 0, 0)),
        out_shape=jax.ShapeDtypeStruct((batch, l, d), jnp.float32),
    )(rows)


def _sc_gather_add(idxp, table, pnp):
    lp = pnp.shape[0]
    d = table.shape[1]
    nq = d // LANES
    batch = idxp.shape[0]
    nblk = batch // NW
    niter = nblk // NBUF

    mesh = plsc.VectorSubcoreMesh(core_axis_name="c", subcore_axis_name="s")

    @functools.partial(
        pl.kernel,
        out_type=jax.ShapeDtypeStruct((batch, lp, d), jnp.float32),
        mesh=mesh,
        scratch_types=[
            pltpu.VMEM((nblk, lp), jnp.int32),
            pltpu.VMEM((lp, d), jnp.float32),
            pltpu.VMEM((NBUF, lp, d), jnp.float32),
            pltpu.VMEM((NBUF, lp, d), jnp.float32),
            pltpu.SemaphoreType.DMA((NBUF,)),
            pltpu.SemaphoreType.DMA((NBUF,)),
        ],
        compiler_params=pltpu.CompilerParams(use_tc_tiling_on_sc=False),
    )
    def k(idx_hbm, table_hbm, pn_hbm, out_hbm, idx_v, pn_v, gbuf, obuf,
          gsem, osem):
        wid = lax.axis_index("s") * NC + lax.axis_index("c")
        blk0 = wid * nblk
        pltpu.sync_copy(idx_hbm.at[pl.ds(blk0, nblk)], idx_v)
        pltpu.sync_copy(pn_hbm, pn_v)

        for s in range(NBUF):
            pltpu.async_copy(table_hbm.at[idx_v.at[s]],
                             gbuf.at[s], gsem.at[s])

        def outer(i, carry):
            base = i * NBUF
            for half in range(2):
                slots = [half * K + j for j in range(K)]
                for s in slots:
                    pltpu.make_async_copy(
                        table_hbm.at[idx_v.at[0]],
                        gbuf.at[s], gsem.at[s]).wait()

                    @pl.when(i > 0)
                    def _(s=s):
                        pltpu.make_async_copy(
                            obuf.at[s], out_hbm.at[0],
                            osem.at[s]).wait()

                def row_body(r, c2):
                    for q in range(nq):
                        col = q * LANES
                        pnq = pn_v[r, pl.ds(col, LANES)]
                        for s in slots:
                            obuf[s, r, pl.ds(col, LANES)] = (
                                gbuf[s, r, pl.ds(col, LANES)] + pnq)
                    return c2

                lax.fori_loop(0, lp, row_body, 0, unroll=1)

                for j, s in enumerate(slots):
                    b = base + half * K + j
                    pltpu.async_copy(
                        obuf.at[s], out_hbm.at[blk0 + b],
                        osem.at[s])

                    @pl.when(i < niter - 1)
                    def _(b=b, s=s):
                        pltpu.async_copy(
                            table_hbm.at[idx_v.at[b + NBUF]],
                            gbuf.at[s], gsem.at[s])
            return carry

        lax.fori_loop(0, niter, outer, 0)

        for s in range(NBUF):
            pltpu.make_async_copy(obuf.at[s], out_hbm.at[0],
                                  osem.at[s]).wait()

    return k(idxp, table, pnp)


def kernel(indices, table, prompt_no):
    batch, l = indices.shape
    d = table.shape[1]
    lp = (l + 7) // 8 * 8
    pnp, idxp = _tc_prep(prompt_no, indices.astype(jnp.int32), lp,
                         table.shape[0])
    rows = _sc_gather_add(idxp, table, pnp)
    return _tc_cast(rows, batch, l, lp, d)


# speedup vs baseline: 1.1484x; 1.1484x over previous
import functools

import jax
import jax.numpy as jnp
from jax import lax
from jax.experimental import pallas as pl
from jax.experimental.pallas import tpu as pltpu
from jax.experimental.pallas import tpu_sc as plsc

NC = 2
NS = 16
NW = NC * NS
LANES = 16
NBUF = 8
K = 4


def _tc_prep(prompt_no, indices, lp, vocab):
    n, l, d = prompt_no.shape
    batch = indices.shape[0]

    def body(p_ref, i_ref, pn_ref, ip_ref):
        pn = jnp.mean(p_ref[...], axis=0)
        pn_ref[...] = jnp.concatenate(
            [pn, jnp.zeros((lp - l, d), jnp.float32)], axis=0)
        pad = (lax.broadcasted_iota(jnp.int32, (batch, lp - l), 0)
               * (lp - l)
               + lax.broadcasted_iota(jnp.int32, (batch, lp - l), 1))
        ip_ref[...] = jnp.concatenate(
            [i_ref[...], (pad * 127) % vocab], axis=1)

    return pl.pallas_call(
        body,
        out_shape=(
            jax.ShapeDtypeStruct((lp, d), jnp.float32),
            jax.ShapeDtypeStruct((batch, lp), jnp.int32),
        ),
    )(prompt_no, indices)


def _tc_cast(rows, batch, l, lp, d):
    bb = 128
    nt = lp * d // 128

    def body(r_ref, o_ref):
        x = r_ref[...]
        x = jnp.transpose(x, (1, 2, 0))
        x = x.reshape(nt, 2, d, bb)
        x = x.reshape(lp, d, bb)
        o_ref[...] = x[:l]

    return pl.pallas_call(
        body,
        grid=(batch // bb,),
        in_specs=[pl.BlockSpec((bb, nt, 128), lambda i: (i, 0, 0))],
        out_specs=pl.BlockSpec((l, d, bb), lambda i: (0, 0, i)),
        out_shape=jax.ShapeDtypeStruct((l, d, batch), jnp.float32),
    )(rows)


def _sc_gather_add(idxp, table, pnp):
    lp = pnp.shape[0]
    d = table.shape[1]
    nq = d // LANES
    batch = idxp.shape[0]
    nblk = batch // NW
    niter = nblk // NBUF
    nt = lp * d // 128

    mesh = plsc.VectorSubcoreMesh(core_axis_name="c", subcore_axis_name="s")

    @functools.partial(
        pl.kernel,
        out_type=jax.ShapeDtypeStruct((batch, nt, 128), jnp.float32),
        mesh=mesh,
        scratch_types=[
            pltpu.VMEM((nblk, lp), jnp.int32),
            pltpu.VMEM((lp, d), jnp.float32),
            pltpu.VMEM((NBUF, lp, d), jnp.float32),
            pltpu.VMEM((NBUF, nt, 128), jnp.float32),
            pltpu.SemaphoreType.DMA((NBUF,)),
            pltpu.SemaphoreType.DMA((NBUF,)),
        ],
        compiler_params=pltpu.CompilerParams(use_tc_tiling_on_sc=False),
    )
    def k(idx_hbm, table_hbm, pn_hbm, out_hbm, idx_v, pn_v, gbuf, obuf,
          gsem, osem):
        wid = lax.axis_index("s") * NC + lax.axis_index("c")
        blk0 = wid * nblk
        pltpu.sync_copy(idx_hbm.at[pl.ds(blk0, nblk)], idx_v)
        pltpu.sync_copy(pn_hbm, pn_v)

        for s in range(NBUF):
            pltpu.async_copy(table_hbm.at[idx_v.at[s]],
                             gbuf.at[s], gsem.at[s])

        def outer(i, carry):
            base = i * NBUF
            for half in range(2):
                slots = [half * K + j for j in range(K)]
                for s in slots:
                    pltpu.make_async_copy(
                        table_hbm.at[idx_v.at[0]],
                        gbuf.at[s], gsem.at[s]).wait()

                    @pl.when(i > 0)
                    def _(s=s):
                        pltpu.make_async_copy(
                            obuf.at[s], out_hbm.at[0],
                            osem.at[s]).wait()

                def row_body(r, c2):
                    for q in range(nq):
                        col = q * LANES
                        t = r * nq + q
                        pnq = pn_v[r, pl.ds(col, LANES)]
                        for s in slots:
                            obuf[s, t >> 3, pl.ds((t & 7) * LANES,
                                                  LANES)] = (
                                gbuf[s, r, pl.ds(col, LANES)] + pnq)
                    return c2

                lax.fori_loop(0, lp, row_body, 0, unroll=1)

                for j, s in enumerate(slots):
                    b = base + half * K + j
                    pltpu.async_copy(
                        obuf.at[s], out_hbm.at[blk0 + b],
                        osem.at[s])

                    @pl.when(i < niter - 1)
                    def _(b=b, s=s):
                        pltpu.async_copy(
                            table_hbm.at[idx_v.at[b + NBUF]],
                            gbuf.at[s], gsem.at[s])
            return carry

        lax.fori_loop(0, niter, outer, 0)

        for s in range(NBUF):
            pltpu.make_async_copy(obuf.at[s], out_hbm.at[0],
                                  osem.at[s]).wait()

    return k(idxp, table, pnp)


def kernel(indices, table, prompt_no):
    batch, l = indices.shape
    d = table.shape[1]
    lp = (l + 7) // 8 * 8
    pnp, idxp = _tc_prep(prompt_no, indices.astype(jnp.int32), lp,
                         table.shape[0])
    rows = _sc_gather_add(idxp, table, pnp)
    out_t = _tc_cast(rows, batch, l, lp, d)
    return jnp.transpose(out_t, (2, 0, 1))


# speedup vs baseline: 1.3449x; 1.1711x over previous
import functools

import jax
import jax.numpy as jnp
from jax import lax
from jax.experimental import pallas as pl
from jax.experimental.pallas import tpu as pltpu
from jax.experimental.pallas import tpu_sc as plsc

NC = 2
NS = 16
NW = NC * NS
LANES = 16
NBUF = 8
K = 4


def _tc_prep(prompt_no, indices, lp, vocab):
    n, l, d = prompt_no.shape
    batch = indices.shape[0]

    def body(p_ref, i_ref, pn_ref, ip_ref):
        pn = jnp.mean(p_ref[...], axis=0)
        pn_ref[...] = jnp.concatenate(
            [pn, jnp.zeros((lp - l, d), jnp.float32)], axis=0)
        pad = (lax.broadcasted_iota(jnp.int32, (batch, lp - l), 0)
               * (lp - l)
               + lax.broadcasted_iota(jnp.int32, (batch, lp - l), 1))
        ip_ref[...] = jnp.concatenate(
            [i_ref[...], (pad * 127) % vocab], axis=1)

    return pl.pallas_call(
        body,
        out_shape=(
            jax.ShapeDtypeStruct((lp, d), jnp.float32),
            jax.ShapeDtypeStruct((batch, lp), jnp.int32),
        ),
    )(prompt_no, indices)


def _tc_cast(rows, batch, l, lp, d):
    bb = 128
    nt = lp * d // 128

    def body(r_ref, o_ref):
        x = r_ref[...]
        x = jnp.transpose(x, (1, 2, 0))
        x = x.reshape(nt, 2, d, bb)
        x = x.reshape(lp, d, bb)
        o_ref[...] = x[:l]

    return pl.pallas_call(
        body,
        grid=(batch // bb,),
        in_specs=[pl.BlockSpec((bb, nt, 128), lambda i: (i, 0, 0))],
        out_specs=pl.BlockSpec((l, d, bb), lambda i: (0, 0, i)),
        out_shape=jax.ShapeDtypeStruct((l, d, batch), jnp.float32),
    )(rows)


def _sc_gather_add(idxp, table, pnp):
    lp = pnp.shape[0]
    d = table.shape[1]
    nq = d // LANES
    batch = idxp.shape[0]
    nblk = batch // NW
    niter = nblk // NBUF
    nt = lp * d // 128

    mesh = plsc.VectorSubcoreMesh(core_axis_name="c", subcore_axis_name="s")

    @functools.partial(
        pl.kernel,
        out_type=jax.ShapeDtypeStruct((batch, nt, 128), jnp.float32),
        mesh=mesh,
        scratch_types=[
            pltpu.VMEM((nblk, lp), jnp.int32),
            pltpu.VMEM((lp, d), jnp.float32),
            pltpu.VMEM((NBUF, lp, d), jnp.float32),
            pltpu.VMEM((NBUF, nt, 128), jnp.float32),
            pltpu.SemaphoreType.DMA((NBUF,)),
            pltpu.SemaphoreType.DMA((NBUF,)),
        ],
        compiler_params=pltpu.CompilerParams(use_tc_tiling_on_sc=False),
    )
    def k(idx_hbm, table_hbm, pn_hbm, out_hbm, idx_v, pn_v, gbuf, obuf,
          gsem, osem):
        wid = lax.axis_index("s") * NC + lax.axis_index("c")
        blk0 = wid * nblk
        pltpu.sync_copy(idx_hbm.at[pl.ds(blk0, nblk)], idx_v)
        pltpu.sync_copy(pn_hbm, pn_v)

        for s in range(NBUF):
            pltpu.async_copy(table_hbm.at[idx_v.at[s]],
                             gbuf.at[s], gsem.at[s])

        def outer(i, carry):
            base = i * NBUF
            for half in range(2):
                slots = [half * K + j for j in range(K)]
                for s in slots:
                    pltpu.make_async_copy(
                        table_hbm.at[idx_v.at[0]],
                        gbuf.at[s], gsem.at[s]).wait()

                    @pl.when(i > 0)
                    def _(s=s):
                        pltpu.make_async_copy(
                            obuf.at[s], out_hbm.at[0],
                            osem.at[s]).wait()

                def row_body(u, c2):
                    for uq in range(128 // LANES):
                        r = u * 2 + uq // nq
                        col = (uq % nq) * LANES
                        pnq = pn_v[r, pl.ds(col, LANES)]
                        for s in slots:
                            obuf[s, u, pl.ds(uq * LANES, LANES)] = (
                                gbuf[s, r, pl.ds(col, LANES)] + pnq)
                    return c2

                lax.fori_loop(0, nt, row_body, 0, unroll=1)

                for j, s in enumerate(slots):
                    b = base + half * K + j
                    pltpu.async_copy(
                        obuf.at[s], out_hbm.at[blk0 + b],
                        osem.at[s])

                    @pl.when(i < niter - 1)
                    def _(b=b, s=s):
                        pltpu.async_copy(
                            table_hbm.at[idx_v.at[b + NBUF]],
                            gbuf.at[s], gsem.at[s])
            return carry

        lax.fori_loop(0, niter, outer, 0)

        for s in range(NBUF):
            pltpu.make_async_copy(obuf.at[s], out_hbm.at[0],
                                  osem.at[s]).wait()

    return k(idxp, table, pnp)


def kernel(indices, table, prompt_no):
    batch, l = indices.shape
    d = table.shape[1]
    lp = (l + 7) // 8 * 8
    pnp, idxp = _tc_prep(prompt_no, indices.astype(jnp.int32), lp,
                         table.shape[0])
    rows = _sc_gather_add(idxp, table, pnp)
    out_t = _tc_cast(rows, batch, l, lp, d)
    return jnp.transpose(out_t, (2, 0, 1))
